# baseline (speedup 1.0000x reference)
import functools

import jax
import jax.numpy as jnp
from jax import lax
from jax.experimental import pallas as pl
from jax.experimental.pallas import tpu as pltpu
from jax.experimental.pallas import tpu_sc as plsc

_CHUNK = 128
_WIN = 16
_NW = 32
_BLK = 1000


def _fill(ref, value, width):
    def _row(i, carry):
        for c in range(width // 16):
            ref[i, pl.ds(c * 16, 16)] = jnp.full((16,), value, jnp.float32)
        return carry

    lax.fori_loop(0, _CHUNK, _row, 0)


def _row_blocks(rpt):
    off = 0
    while off < rpt:
        sz = min(_CHUNK, rpt - off)
        yield off, sz
        off += sz


_MESH = plsc.VectorSubcoreMesh(
    core_axis_name="c", subcore_axis_name="s", num_cores=2, num_subcores=16)


def _make_sc_agg(n_acc, width, nch):
    rpt = n_acc // 16

    nwin = nch // _WIN
    assert nch % _WIN == 0

    @functools.partial(
        pl.kernel,
        out_type=jax.ShapeDtypeStruct((2, n_acc, width), jnp.float32),
        mesh=_MESH,
        scratch_types=[
            pltpu.VMEM((2 * _WIN, _CHUNK), jnp.int32),
            pltpu.VMEM((2 * _WIN, _CHUNK), jnp.int32),
            pltpu.VMEM((_CHUNK, width), jnp.float32),
            pltpu.VMEM((_CHUNK, width), jnp.float32),
            pltpu.VMEM_SHARED((n_acc, width), jnp.float32),
            pltpu.SemaphoreType.DMA,
            pltpu.SemaphoreType.DMA,
            pltpu.SemaphoreType.DMA,
        ],
    )
    def sc_agg(table, src_idx, dst_idx, out,
               src_w, dst_w, rows_a, rows_b, acc, sem_a, sem_b, sem_w):
        cid = lax.axis_index("c")
        sid = lax.axis_index("s")
        wid = sid * 2 + cid
        base = sid * rpt

        _fill(rows_a, 0.0, width)
        for off, sz in _row_blocks(rpt):
            pltpu.sync_copy(rows_a.at[pl.ds(0, sz)],
                            acc.at[pl.ds(base + off, sz)])
        plsc.subcore_barrier()

        def _fetch_window(w, slot):
            o = slot * _WIN
            return (
                pltpu.make_async_copy(src_idx.at[wid, pl.ds(w * _WIN, _WIN)],
                                      src_w.at[pl.ds(o, _WIN)], sem_w),
                pltpu.make_async_copy(dst_idx.at[wid, pl.ds(w * _WIN, _WIN)],
                                      dst_w.at[pl.ds(o, _WIN)], sem_w),
            )

        for d in _fetch_window(0, 0):
            d.start()

        def _window(w, carry):
            slot = w % 2
            o = slot * _WIN
            for d in _fetch_window(w, slot):
                d.wait()

            @pl.when(w + 1 < nwin)
            def _():
                for d in _fetch_window(w + 1, 1 - slot):
                    d.start()

            pltpu.async_copy(table.at[src_w.at[o]], rows_a, sem_a)

            def _pair(i, c_):
                k = o + 2 * i
                pltpu.async_copy(table.at[src_w.at[k + 1]], rows_b, sem_b)
                pltpu.make_async_copy(
                    table.at[src_w.at[k]], rows_a, sem_a).wait()
                pltpu.sync_copy(rows_a, acc.at[dst_w.at[k]], add=True)

                @pl.when(2 * i + 2 < _WIN)
                def _():
                    pltpu.async_copy(table.at[src_w.at[k + 2]], rows_a, sem_a)

                pltpu.make_async_copy(
                    table.at[src_w.at[k + 1]], rows_b, sem_b).wait()
                pltpu.sync_copy(rows_b, acc.at[dst_w.at[k + 1]], add=True)
                return c_

            lax.fori_loop(0, _WIN // 2, _pair, 0)
            return carry

        lax.fori_loop(0, nwin, _window, 0)
        plsc.subcore_barrier()

        for off, sz in _row_blocks(rpt):
            pltpu.sync_copy(acc.at[pl.ds(base + off, sz)],
                            rows_a.at[pl.ds(0, sz)])
            pltpu.sync_copy(rows_a.at[pl.ds(0, sz)],
                            out.at[cid, pl.ds(base + off, sz)])

    return sc_agg


def _make_sc_deg(n_acc, nch, dw=128):
    rpt = n_acc // 16

    @functools.partial(
        pl.kernel,
        out_type=jax.ShapeDtypeStruct((2, n_acc, dw), jnp.float32),
        mesh=_MESH,
        scratch_types=[
            pltpu.VMEM((nch, _CHUNK), jnp.int32),
            pltpu.VMEM((_CHUNK, dw), jnp.float32),
            pltpu.VMEM_SHARED((n_acc, dw), jnp.float32),
        ],
    )
    def sc_deg(dst_idx, out, dst_v, ones_v, dacc):
        cid = lax.axis_index("c")
        sid = lax.axis_index("s")
        wid = sid * 2 + cid
        base = sid * rpt

        _fill(ones_v, 0.0, dw)
        for off, sz in _row_blocks(rpt):
            pltpu.sync_copy(ones_v.at[pl.ds(0, sz)],
                            dacc.at[pl.ds(base + off, sz)])
        _fill(ones_v, 1.0, dw)
        plsc.subcore_barrier()

        pltpu.sync_copy(dst_idx.at[wid], dst_v)

        def _edge_chunk(j, carry):
            pltpu.sync_copy(ones_v, dacc.at[dst_v.at[j]], add=True)
            return carry

        lax.fori_loop(0, nch, _edge_chunk, 0)
        plsc.subcore_barrier()

        for off, sz in _row_blocks(rpt):
            pltpu.sync_copy(dacc.at[pl.ds(base + off, sz)],
                            ones_v.at[pl.ds(0, sz)])
            pltpu.sync_copy(ones_v.at[pl.ds(0, sz)],
                            out.at[cid, pl.ds(base + off, sz)])

    return sc_deg


def _table1_body(x_ref, w_ref, o_ref):
    o_ref[...] = jnp.dot(
        x_ref[...], w_ref[...], preferred_element_type=jnp.float32)


def _mid_body(x_ref, acc_ref, deg_ref, ws_ref, b_ref, wn2_ref,
              h1_ref, m2_ref, rd_ref):
    agg = acc_ref[0] + acc_ref[1]
    deg = (deg_ref[0] + deg_ref[1])[:, 0:1]
    rdeg = 1.0 / jnp.maximum(deg, 1.0)
    h = jnp.dot(x_ref[...], ws_ref[...], preferred_element_type=jnp.float32)
    h = jnp.maximum(h + agg * rdeg + b_ref[...], 0.0)
    h1_ref[...] = h
    m2_ref[...] = jnp.dot(h, wn2_ref[...], preferred_element_type=jnp.float32)
    rd_ref[...] = jnp.broadcast_to(rdeg, rd_ref.shape)


def _out_body(h1_ref, acc_ref, rd_ref, ws_ref, b_ref, o_ref):
    s = acc_ref[0] + acc_ref[1]
    o = jnp.dot(h1_ref[...], ws_ref[...], preferred_element_type=jnp.float32)
    o_ref[...] = o + s * rd_ref[...] + b_ref[...]


def kernel(x, edge_index, W_self1, W_neigh1, b1, W_self2, W_neigh2, b2):
    N, D = x.shape
    H = W_self1.shape[1]
    E = edge_index.shape[1]
    nch = -(-(-(-E // (_NW * _CHUNK))) // _WIN) * _WIN
    e_pad = nch * _NW * _CHUNK
    n_acc = -(-(N + 1) // 2048) * 2048
    grid = N // _BLK

    src = edge_index[0]
    dst = edge_index[1]
    pad = e_pad - E
    src_r = jnp.concatenate([src, jnp.zeros((pad,), jnp.int32)]).reshape(
        _NW, nch, _CHUNK)
    pad_dst = N + (jnp.arange(pad, dtype=jnp.int32) % (n_acc - N))
    dst_r = jnp.concatenate([dst, pad_dst]).reshape(_NW, nch, _CHUNK)

    m1 = pl.pallas_call(
        _table1_body,
        grid=(grid,),
        in_specs=[
            pl.BlockSpec((_BLK, D), lambda i: (i, 0)),
            pl.BlockSpec((D, H), lambda i: (0, 0)),
        ],
        out_specs=pl.BlockSpec((_BLK, H), lambda i: (i, 0)),
        out_shape=jax.ShapeDtypeStruct((N, H), jnp.float32),
    )(x, W_neigh1)

    deg1 = _make_sc_deg(n_acc, nch)(dst_r)
    acc1 = _make_sc_agg(n_acc, H, nch)(m1, src_r, dst_r)

    h1, m2, rdeg = pl.pallas_call(
        _mid_body,
        grid=(grid,),
        in_specs=[
            pl.BlockSpec((_BLK, D), lambda i: (i, 0)),
            pl.BlockSpec((2, _BLK, H), lambda i: (0, i, 0)),
            pl.BlockSpec((2, _BLK, 128), lambda i: (0, i, 0)),
            pl.BlockSpec((D, H), lambda i: (0, 0)),
            pl.BlockSpec((1, H), lambda i: (0, 0)),
            pl.BlockSpec((H, H), lambda i: (0, 0)),
        ],
        out_specs=[
            pl.BlockSpec((_BLK, H), lambda i: (i, 0)),
            pl.BlockSpec((_BLK, H), lambda i: (i, 0)),
            pl.BlockSpec((_BLK, H), lambda i: (i, 0)),
        ],
        out_shape=[
            jax.ShapeDtypeStruct((N, H), jnp.float32),
            jax.ShapeDtypeStruct((N, H), jnp.float32),
            jax.ShapeDtypeStruct((N, H), jnp.float32),
        ],
    )(x, acc1, deg1, W_self1, b1.reshape(1, H), W_neigh2)

    acc2 = _make_sc_agg(n_acc, H, nch)(m2, src_r, dst_r)

    out = pl.pallas_call(
        _out_body,
        grid=(grid,),
        in_specs=[
            pl.BlockSpec((_BLK, H), lambda i: (i, 0)),
            pl.BlockSpec((2, _BLK, H), lambda i: (0, i, 0)),
            pl.BlockSpec((_BLK, H), lambda i: (i, 0)),
            pl.BlockSpec((H, H), lambda i: (0, 0)),
            pl.BlockSpec((1, H), lambda i: (0, 0)),
        ],
        out_specs=pl.BlockSpec((_BLK, H), lambda i: (i, 0)),
        out_shape=jax.ShapeDtypeStruct((N, H), jnp.float32),
    )(h1, acc2, rdeg, W_self2, b2.reshape(1, H))
    return out


# speedup vs baseline: 1.3336x; 1.3336x over previous
import functools

import jax
import jax.numpy as jnp
from jax import lax
from jax.experimental import pallas as pl
from jax.experimental.pallas import tpu as pltpu
from jax.experimental.pallas import tpu_sc as plsc

_CHUNK = 128
_WIN = 16
_NW = 32
_BLK = 1000


def _fill(ref, value, width):
    def _row(i, carry):
        for c in range(width // 16):
            ref[i, pl.ds(c * 16, 16)] = jnp.full((16,), value, jnp.float32)
        return carry

    lax.fori_loop(0, _CHUNK, _row, 0)


def _row_blocks(rpt):
    off = 0
    while off < rpt:
        sz = min(_CHUNK, rpt - off)
        yield off, sz
        off += sz


_MESH = plsc.VectorSubcoreMesh(
    core_axis_name="c", subcore_axis_name="s", num_cores=2, num_subcores=16)


def _make_sc_agg(n_acc, width, nch):
    rpt = n_acc // 16

    @functools.partial(
        pl.kernel,
        out_type=jax.ShapeDtypeStruct((2, n_acc, width), jnp.float32),
        mesh=_MESH,
        scratch_types=[
            pltpu.VMEM((nch, _CHUNK), jnp.int32),
            pltpu.VMEM((nch, _CHUNK), jnp.int32),
            pltpu.VMEM((_CHUNK, width), jnp.float32),
            pltpu.VMEM_SHARED((n_acc, width), jnp.float32),
            pltpu.SemaphoreType.DMA,
        ],
    )
    def sc_agg(table, src_idx, dst_idx, out,
               src_v, dst_v, rows_a, acc, sem_a):
        cid = lax.axis_index("c")
        sid = lax.axis_index("s")
        wid = sid * 2 + cid
        base = sid * rpt

        _fill(rows_a, 0.0, width)
        for off, sz in _row_blocks(rpt):
            pltpu.sync_copy(rows_a.at[pl.ds(0, sz)],
                            acc.at[pl.ds(base + off, sz)])
        plsc.subcore_barrier()

        pltpu.sync_copy(src_idx.at[wid], src_v)
        pltpu.sync_copy(dst_idx.at[wid], dst_v)

        def _chunk(j, c_):
            pltpu.async_copy(table.at[src_v.at[j]], rows_a, sem_a).wait()
            pltpu.sync_copy(rows_a, acc.at[dst_v.at[j]], add=True)
            return c_

        lax.fori_loop(0, nch, _chunk, 0)
        plsc.subcore_barrier()

        for off, sz in _row_blocks(rpt):
            pltpu.sync_copy(acc.at[pl.ds(base + off, sz)],
                            rows_a.at[pl.ds(0, sz)])
            pltpu.sync_copy(rows_a.at[pl.ds(0, sz)],
                            out.at[cid, pl.ds(base + off, sz)])

    return sc_agg


def _make_sc_deg(n_acc, nch, dw=128):
    rpt = n_acc // 16

    @functools.partial(
        pl.kernel,
        out_type=jax.ShapeDtypeStruct((2, n_acc, dw), jnp.float32),
        mesh=_MESH,
        scratch_types=[
            pltpu.VMEM((nch, _CHUNK), jnp.int32),
            pltpu.VMEM((_CHUNK, dw), jnp.float32),
            pltpu.VMEM_SHARED((n_acc, dw), jnp.float32),
        ],
    )
    def sc_deg(dst_idx, out, dst_v, ones_v, dacc):
        cid = lax.axis_index("c")
        sid = lax.axis_index("s")
        wid = sid * 2 + cid
        base = sid * rpt

        _fill(ones_v, 0.0, dw)
        for off, sz in _row_blocks(rpt):
            pltpu.sync_copy(ones_v.at[pl.ds(0, sz)],
                            dacc.at[pl.ds(base + off, sz)])
        _fill(ones_v, 1.0, dw)
        plsc.subcore_barrier()

        pltpu.sync_copy(dst_idx.at[wid], dst_v)

        def _edge_chunk(j, carry):
            pltpu.sync_copy(ones_v, dacc.at[dst_v.at[j]], add=True)
            return carry

        lax.fori_loop(0, nch, _edge_chunk, 0)
        plsc.subcore_barrier()

        for off, sz in _row_blocks(rpt):
            pltpu.sync_copy(dacc.at[pl.ds(base + off, sz)],
                            ones_v.at[pl.ds(0, sz)])
            pltpu.sync_copy(ones_v.at[pl.ds(0, sz)],
                            out.at[cid, pl.ds(base + off, sz)])

    return sc_deg


def _table1_body(x_ref, w_ref, o_ref):
    o_ref[...] = jnp.dot(
        x_ref[...], w_ref[...], preferred_element_type=jnp.float32)


def _mid_body(x_ref, acc_ref, deg_ref, ws_ref, b_ref, wn2_ref,
              h1_ref, m2_ref, rd_ref):
    agg = acc_ref[0] + acc_ref[1]
    deg = (deg_ref[0] + deg_ref[1])[:, 0:1]
    rdeg = 1.0 / jnp.maximum(deg, 1.0)
    h = jnp.dot(x_ref[...], ws_ref[...], preferred_element_type=jnp.float32)
    h = jnp.maximum(h + agg * rdeg + b_ref[...], 0.0)
    h1_ref[...] = h
    m2_ref[...] = jnp.dot(h, wn2_ref[...], preferred_element_type=jnp.float32)
    rd_ref[...] = jnp.broadcast_to(rdeg, rd_ref.shape)


def _out_body(h1_ref, acc_ref, rd_ref, ws_ref, b_ref, o_ref):
    s = acc_ref[0] + acc_ref[1]
    o = jnp.dot(h1_ref[...], ws_ref[...], preferred_element_type=jnp.float32)
    o_ref[...] = o + s * rd_ref[...] + b_ref[...]


def kernel(x, edge_index, W_self1, W_neigh1, b1, W_self2, W_neigh2, b2):
    N, D = x.shape
    H = W_self1.shape[1]
    E = edge_index.shape[1]
    nch = -(-E // (_NW * _CHUNK))
    e_pad = nch * _NW * _CHUNK
    n_acc = -(-(N + 1) // 2048) * 2048
    grid = N // _BLK

    src = edge_index[0]
    dst = edge_index[1]
    pad = e_pad - E
    src_r = jnp.concatenate([src, jnp.zeros((pad,), jnp.int32)]).reshape(
        _NW, nch, _CHUNK)
    pad_dst = N + (jnp.arange(pad, dtype=jnp.int32) % (n_acc - N))
    dst_r = jnp.concatenate([dst, pad_dst]).reshape(_NW, nch, _CHUNK)

    m1 = pl.pallas_call(
        _table1_body,
        grid=(grid,),
        in_specs=[
            pl.BlockSpec((_BLK, D), lambda i: (i, 0)),
            pl.BlockSpec((D, H), lambda i: (0, 0)),
        ],
        out_specs=pl.BlockSpec((_BLK, H), lambda i: (i, 0)),
        out_shape=jax.ShapeDtypeStruct((N, H), jnp.float32),
    )(x, W_neigh1)

    deg1 = _make_sc_deg(n_acc, nch)(dst_r)
    acc1 = _make_sc_agg(n_acc, H, nch)(m1, src_r, dst_r)

    h1, m2, rdeg = pl.pallas_call(
        _mid_body,
        grid=(grid,),
        in_specs=[
            pl.BlockSpec((_BLK, D), lambda i: (i, 0)),
            pl.BlockSpec((2, _BLK, H), lambda i: (0, i, 0)),
            pl.BlockSpec((2, _BLK, 128), lambda i: (0, i, 0)),
            pl.BlockSpec((D, H), lambda i: (0, 0)),
            pl.BlockSpec((1, H), lambda i: (0, 0)),
            pl.BlockSpec((H, H), lambda i: (0, 0)),
        ],
        out_specs=[
            pl.BlockSpec((_BLK, H), lambda i: (i, 0)),
            pl.BlockSpec((_BLK, H), lambda i: (i, 0)),
            pl.BlockSpec((_BLK, H), lambda i: (i, 0)),
        ],
        out_shape=[
            jax.ShapeDtypeStruct((N, H), jnp.float32),
            jax.ShapeDtypeStruct((N, H), jnp.float32),
            jax.ShapeDtypeStruct((N, H), jnp.float32),
        ],
    )(x, acc1, deg1, W_self1, b1.reshape(1, H), W_neigh2)

    acc2 = _make_sc_agg(n_acc, H, nch)(m2, src_r, dst_r)

    out = pl.pallas_call(
        _out_body,
        grid=(grid,),
        in_specs=[
            pl.BlockSpec((_BLK, H), lambda i: (i, 0)),
            pl.BlockSpec((2, _BLK, H), lambda i: (0, i, 0)),
            pl.BlockSpec((_BLK, H), lambda i: (i, 0)),
            pl.BlockSpec((H, H), lambda i: (0, 0)),
            pl.BlockSpec((1, H), lambda i: (0, 0)),
        ],
        out_specs=pl.BlockSpec((_BLK, H), lambda i: (i, 0)),
        out_shape=jax.ShapeDtypeStruct((N, H), jnp.float32),
    )(h1, acc2, rdeg, W_self2, b2.reshape(1, H))
    return out


# speedup vs baseline: 1.4928x; 1.1194x over previous
import functools

import jax
import jax.numpy as jnp
from jax import lax
from jax.experimental import pallas as pl
from jax.experimental.pallas import tpu as pltpu
from jax.experimental.pallas import tpu_sc as plsc

_CHUNK = 128
_FRAC_A = 0.646
_WIN = 16
_NW = 32
_BLK = 1000


def _fill(ref, value, width):
    def _row(i, carry):
        for c in range(width // 16):
            ref[i, pl.ds(c * 16, 16)] = jnp.full((16,), value, jnp.float32)
        return carry

    lax.fori_loop(0, _CHUNK, _row, 0)


def _row_blocks(rpt):
    off = 0
    while off < rpt:
        sz = min(_CHUNK, rpt - off)
        yield off, sz
        off += sz


_MESH = plsc.VectorSubcoreMesh(
    core_axis_name="c", subcore_axis_name="s", num_cores=2, num_subcores=16)


def _make_sc_agg(n_acc, width, nch_a, nch_b):
    rpt = n_acc // 16
    nch = max(nch_a, nch_b)

    @functools.partial(
        pl.kernel,
        out_type=jax.ShapeDtypeStruct((2, n_acc, width), jnp.float32),
        mesh=_MESH,
        scratch_types=[
            pltpu.VMEM((nch, _CHUNK), jnp.int32),
            pltpu.VMEM((nch, _CHUNK), jnp.int32),
            pltpu.VMEM((_CHUNK, width), jnp.float32),
            pltpu.VMEM_SHARED((n_acc, width), jnp.float32),
            pltpu.SemaphoreType.DMA,
        ],
    )
    def sc_agg(table, src_a, dst_a, src_b, dst_b, out,
               src_v, dst_v, rows_a, acc, sem_a):
        cid = lax.axis_index("c")
        sid = lax.axis_index("s")
        base = sid * rpt

        _fill(rows_a, 0.0, width)
        for off, sz in _row_blocks(rpt):
            pltpu.sync_copy(rows_a.at[pl.ds(0, sz)],
                            acc.at[pl.ds(base + off, sz)])
        plsc.subcore_barrier()

        @pl.when(cid == 0)
        def _():
            pltpu.sync_copy(src_a.at[sid], src_v.at[pl.ds(0, nch_a)])
            pltpu.sync_copy(dst_a.at[sid], dst_v.at[pl.ds(0, nch_a)])

        @pl.when(cid == 1)
        def _():
            pltpu.sync_copy(src_b.at[sid], src_v.at[pl.ds(0, nch_b)])
            pltpu.sync_copy(dst_b.at[sid], dst_v.at[pl.ds(0, nch_b)])

        def _chunk(j, c_):
            pltpu.async_copy(table.at[src_v.at[j]], rows_a, sem_a).wait()
            pltpu.sync_copy(rows_a, acc.at[dst_v.at[j]], add=True)
            return c_

        lax.fori_loop(0, jnp.where(cid == 0, nch_a, nch_b), _chunk, 0)
        plsc.subcore_barrier()

        for off, sz in _row_blocks(rpt):
            pltpu.sync_copy(acc.at[pl.ds(base + off, sz)],
                            rows_a.at[pl.ds(0, sz)])
            pltpu.sync_copy(rows_a.at[pl.ds(0, sz)],
                            out.at[cid, pl.ds(base + off, sz)])

    return sc_agg


def _make_sc_deg(n_acc, nch, dw=128):
    rpt = n_acc // 16

    @functools.partial(
        pl.kernel,
        out_type=jax.ShapeDtypeStruct((2, n_acc, dw), jnp.float32),
        mesh=_MESH,
        scratch_types=[
            pltpu.VMEM((nch, _CHUNK), jnp.int32),
            pltpu.VMEM((_CHUNK, dw), jnp.float32),
            pltpu.VMEM_SHARED((n_acc, dw), jnp.float32),
        ],
    )
    def sc_deg(dst_idx, out, dst_v, ones_v, dacc):
        cid = lax.axis_index("c")
        sid = lax.axis_index("s")
        wid = sid * 2 + cid
        base = sid * rpt

        _fill(ones_v, 0.0, dw)
        for off, sz in _row_blocks(rpt):
            pltpu.sync_copy(ones_v.at[pl.ds(0, sz)],
                            dacc.at[pl.ds(base + off, sz)])
        _fill(ones_v, 1.0, dw)
        plsc.subcore_barrier()

        pltpu.sync_copy(dst_idx.at[wid], dst_v)

        def _edge_chunk(j, carry):
            pltpu.sync_copy(ones_v, dacc.at[dst_v.at[j]], add=True)
            return carry

        lax.fori_loop(0, nch, _edge_chunk, 0)
        plsc.subcore_barrier()

        for off, sz in _row_blocks(rpt):
            pltpu.sync_copy(dacc.at[pl.ds(base + off, sz)],
                            ones_v.at[pl.ds(0, sz)])
            pltpu.sync_copy(ones_v.at[pl.ds(0, sz)],
                            out.at[cid, pl.ds(base + off, sz)])

    return sc_deg


def _table1_body(x_ref, w_ref, o_ref):
    o_ref[...] = jnp.dot(
        x_ref[...], w_ref[...], preferred_element_type=jnp.float32)


def _mid_body(x_ref, acc_ref, deg_ref, ws_ref, b_ref, wn2_ref,
              h1_ref, m2_ref, rd_ref):
    agg = acc_ref[0] + acc_ref[1]
    deg = (deg_ref[0] + deg_ref[1])[:, 0:1]
    rdeg = 1.0 / jnp.maximum(deg, 1.0)
    h = jnp.dot(x_ref[...], ws_ref[...], preferred_element_type=jnp.float32)
    h = jnp.maximum(h + agg * rdeg + b_ref[...], 0.0)
    h1_ref[...] = h
    m2_ref[...] = jnp.dot(h, wn2_ref[...], preferred_element_type=jnp.float32)
    rd_ref[...] = jnp.broadcast_to(rdeg, rd_ref.shape)


def _out_body(h1_ref, acc_ref, rd_ref, ws_ref, b_ref, o_ref):
    s = acc_ref[0] + acc_ref[1]
    o = jnp.dot(h1_ref[...], ws_ref[...], preferred_element_type=jnp.float32)
    o_ref[...] = o + s * rd_ref[...] + b_ref[...]


def kernel(x, edge_index, W_self1, W_neigh1, b1, W_self2, W_neigh2, b2):
    N, D = x.shape
    H = W_self1.shape[1]
    E = edge_index.shape[1]
    nch = -(-E // (_NW * _CHUNK))
    e_pad = nch * _NW * _CHUNK
    n_acc = -(-(N + 1) // 2048) * 2048
    grid = N // _BLK

    src = edge_index[0]
    dst = edge_index[1]
    pad = e_pad - E
    src_p = jnp.concatenate([src, jnp.zeros((pad,), jnp.int32)])
    pad_dst = N + (jnp.arange(pad, dtype=jnp.int32) % (n_acc - N))
    dst_p = jnp.concatenate([dst, pad_dst])
    dst_r = dst_p.reshape(_NW, nch, _CHUNK)

    tot_ch = _NW * nch
    nch_a = int(round(tot_ch * _FRAC_A / 16))
    nch_b = tot_ch // 16 - nch_a
    cut = 16 * nch_a * _CHUNK
    src_a = src_p[:cut].reshape(16, nch_a, _CHUNK)
    dst_a = dst_p[:cut].reshape(16, nch_a, _CHUNK)
    src_b = src_p[cut:].reshape(16, nch_b, _CHUNK)
    dst_b = dst_p[cut:].reshape(16, nch_b, _CHUNK)

    m1 = pl.pallas_call(
        _table1_body,
        grid=(grid,),
        in_specs=[
            pl.BlockSpec((_BLK, D), lambda i: (i, 0)),
            pl.BlockSpec((D, H), lambda i: (0, 0)),
        ],
        out_specs=pl.BlockSpec((_BLK, H), lambda i: (i, 0)),
        out_shape=jax.ShapeDtypeStruct((N, H), jnp.float32),
    )(x, W_neigh1)

    deg1 = _make_sc_deg(n_acc, nch)(dst_r)
    acc1 = _make_sc_agg(n_acc, H, nch_a, nch_b)(m1, src_a, dst_a, src_b, dst_b)

    h1, m2, rdeg = pl.pallas_call(
        _mid_body,
        grid=(grid,),
        in_specs=[
            pl.BlockSpec((_BLK, D), lambda i: (i, 0)),
            pl.BlockSpec((2, _BLK, H), lambda i: (0, i, 0)),
            pl.BlockSpec((2, _BLK, 128), lambda i: (0, i, 0)),
            pl.BlockSpec((D, H), lambda i: (0, 0)),
            pl.BlockSpec((1, H), lambda i: (0, 0)),
            pl.BlockSpec((H, H), lambda i: (0, 0)),
        ],
        out_specs=[
            pl.BlockSpec((_BLK, H), lambda i: (i, 0)),
            pl.BlockSpec((_BLK, H), lambda i: (i, 0)),
            pl.BlockSpec((_BLK, H), lambda i: (i, 0)),
        ],
        out_shape=[
            jax.ShapeDtypeStruct((N, H), jnp.float32),
            jax.ShapeDtypeStruct((N, H), jnp.float32),
            jax.ShapeDtypeStruct((N, H), jnp.float32),
        ],
    )(x, acc1, deg1, W_self1, b1.reshape(1, H), W_neigh2)

    acc2 = _make_sc_agg(n_acc, H, nch_a, nch_b)(m2, src_a, dst_a, src_b, dst_b)

    out = pl.pallas_call(
        _out_body,
        grid=(grid,),
        in_specs=[
            pl.BlockSpec((_BLK, H), lambda i: (i, 0)),
            pl.BlockSpec((2, _BLK, H), lambda i: (0, i, 0)),
            pl.BlockSpec((_BLK, H), lambda i: (i, 0)),
            pl.BlockSpec((H, H), lambda i: (0, 0)),
            pl.BlockSpec((1, H), lambda i: (0, 0)),
        ],
        out_specs=pl.BlockSpec((_BLK, H), lambda i: (i, 0)),
        out_shape=jax.ShapeDtypeStruct((N, H), jnp.float32),
    )(h1, acc2, rdeg, W_self2, b2.reshape(1, H))
    return out


# speedup vs baseline: 1.8201x; 1.2193x over previous
import functools

import jax
import jax.numpy as jnp
from jax import lax
from jax.experimental import pallas as pl
from jax.experimental.pallas import tpu as pltpu
from jax.experimental.pallas import tpu_sc as plsc

_CHUNK = 128
_FRAC_A = 0.646
_WIN = 16
_NW = 32
_BLK = 1000


def _fill(ref, value, width):
    def _row(i, carry):
        for c in range(width // 16):
            ref[i, pl.ds(c * 16, 16)] = jnp.full((16,), value, jnp.float32)
        return carry

    lax.fori_loop(0, _CHUNK, _row, 0)


def _row_blocks(rpt):
    off = 0
    while off < rpt:
        sz = min(_CHUNK, rpt - off)
        yield off, sz
        off += sz


_MESH = plsc.VectorSubcoreMesh(
    core_axis_name="c", subcore_axis_name="s", num_cores=2, num_subcores=16)


def _make_sc_agg(n_acc, width, nch_a, nch_b):
    rpt = n_acc // 16
    nch = max(nch_a, nch_b)

    @functools.partial(
        pl.kernel,
        out_type=jax.ShapeDtypeStruct((2, n_acc, width), jnp.float32),
        mesh=_MESH,
        scratch_types=[
            pltpu.VMEM((nch, _CHUNK), jnp.int32),
            pltpu.VMEM((nch, _CHUNK), jnp.int32),
            pltpu.VMEM((_CHUNK, width), jnp.float32),
            pltpu.VMEM_SHARED((n_acc, width), jnp.float32),
            pltpu.SemaphoreType.DMA,
        ],
    )
    def sc_agg(table, src_a, dst_a, src_b, dst_b, out,
               src_v, dst_v, rows_a, acc, sem_a):
        cid = lax.axis_index("c")
        sid = lax.axis_index("s")
        base = sid * rpt

        _fill(rows_a, 0.0, width)
        for off, sz in _row_blocks(rpt):
            pltpu.sync_copy(rows_a.at[pl.ds(0, sz)],
                            acc.at[pl.ds(base + off, sz)])
        plsc.subcore_barrier()

        @pl.when(cid == 0)
        def _():
            pltpu.sync_copy(src_a.at[sid], src_v.at[pl.ds(0, nch_a)])
            pltpu.sync_copy(dst_a.at[sid], dst_v.at[pl.ds(0, nch_a)])

        @pl.when(cid == 1)
        def _():
            pltpu.sync_copy(src_b.at[sid], src_v.at[pl.ds(0, nch_b)])
            pltpu.sync_copy(dst_b.at[sid], dst_v.at[pl.ds(0, nch_b)])

        def _chunk(j, c_):
            pltpu.async_copy(table.at[src_v.at[j]], rows_a, sem_a).wait()
            pltpu.sync_copy(rows_a, acc.at[dst_v.at[j]], add=True)
            return c_

        lax.fori_loop(0, jnp.where(cid == 0, nch_a, nch_b), _chunk, 0)
        plsc.subcore_barrier()

        for off, sz in _row_blocks(rpt):
            pltpu.sync_copy(acc.at[pl.ds(base + off, sz)],
                            rows_a.at[pl.ds(0, sz)])
            pltpu.sync_copy(rows_a.at[pl.ds(0, sz)],
                            out.at[cid, pl.ds(base + off, sz)])

    return sc_agg


def _make_sc_deg(n_acc, nch, dw=128):
    rpt = n_acc // 16

    @functools.partial(
        pl.kernel,
        out_type=jax.ShapeDtypeStruct((2, n_acc, dw), jnp.float32),
        mesh=_MESH,
        scratch_types=[
            pltpu.VMEM((nch, _CHUNK), jnp.int32),
            pltpu.VMEM((_CHUNK, dw), jnp.float32),
            pltpu.VMEM_SHARED((n_acc, dw), jnp.float32),
        ],
    )
    def sc_deg(dst_idx, out, dst_v, ones_v, dacc):
        cid = lax.axis_index("c")
        sid = lax.axis_index("s")
        wid = sid * 2 + cid
        base = sid * rpt

        _fill(ones_v, 0.0, dw)
        for off, sz in _row_blocks(rpt):
            pltpu.sync_copy(ones_v.at[pl.ds(0, sz)],
                            dacc.at[pl.ds(base + off, sz)])
        _fill(ones_v, 1.0, dw)
        plsc.subcore_barrier()

        pltpu.sync_copy(dst_idx.at[wid], dst_v)

        def _edge_chunk(j, carry):
            pltpu.sync_copy(ones_v, dacc.at[dst_v.at[j]], add=True)
            return carry

        lax.fori_loop(0, nch, _edge_chunk, 0)
        plsc.subcore_barrier()

        for off, sz in _row_blocks(rpt):
            pltpu.sync_copy(dacc.at[pl.ds(base + off, sz)],
                            ones_v.at[pl.ds(0, sz)])
            pltpu.sync_copy(ones_v.at[pl.ds(0, sz)],
                            out.at[cid, pl.ds(base + off, sz)])

    return sc_deg


def _table1_body(x_ref, w_ref, o_ref):
    o_ref[...] = jnp.dot(
        x_ref[...], w_ref[...], preferred_element_type=jnp.float32)


def _mid_body(x_ref, acc_ref, deg_ref, ws_ref, b_ref, wn2_ref,
              h1_ref, m2_ref, rd_ref):
    agg = acc_ref[0] + acc_ref[1]
    deg = (deg_ref[0] + deg_ref[1])[:, 0:1]
    rdeg = 1.0 / jnp.maximum(deg, 1.0)
    h = jnp.dot(x_ref[...], ws_ref[...], preferred_element_type=jnp.float32)
    h = jnp.maximum(h + agg * rdeg + b_ref[...], 0.0)
    h1_ref[...] = h
    m2_ref[...] = jnp.dot(h, wn2_ref[...], preferred_element_type=jnp.float32)
    rd_ref[...] = jnp.broadcast_to(rdeg, rd_ref.shape)


def _out_body(h1_ref, acc_ref, rd_ref, ws_ref, b_ref, o_ref):
    s = acc_ref[0] + acc_ref[1]
    o = jnp.dot(h1_ref[...], ws_ref[...], preferred_element_type=jnp.float32)
    o_ref[...] = o + s * rd_ref[...] + b_ref[...]


def kernel(x, edge_index, W_self1, W_neigh1, b1, W_self2, W_neigh2, b2):
    N, D = x.shape
    H = W_self1.shape[1]
    E = edge_index.shape[1]
    nch = -(-E // (_NW * _CHUNK))
    e_pad = nch * _NW * _CHUNK
    n_acc = -(-(N + 1) // 2048) * 2048
    grid = N // _BLK

    src = edge_index[0]
    dst = edge_index[1]
    pad = e_pad - E
    pad_src = jnp.arange(pad, dtype=jnp.int32) % N
    src_p = jnp.concatenate([src, pad_src])
    pad_dst = N + (jnp.arange(pad, dtype=jnp.int32) % (n_acc - N))
    dst_p = jnp.concatenate([dst, pad_dst])
    dst_r = dst_p.reshape(_NW, nch, _CHUNK)

    tot_ch = _NW * nch
    nch_a = int(round(tot_ch * _FRAC_A / 16))
    nch_b = tot_ch // 16 - nch_a
    cut = 16 * nch_a * _CHUNK
    src_a = src_p[:cut].reshape(16, nch_a, _CHUNK)
    dst_a = dst_p[:cut].reshape(16, nch_a, _CHUNK)
    src_b = src_p[cut:].reshape(16, nch_b, _CHUNK)
    dst_b = dst_p[cut:].reshape(16, nch_b, _CHUNK)

    m1 = pl.pallas_call(
        _table1_body,
        grid=(grid,),
        in_specs=[
            pl.BlockSpec((_BLK, D), lambda i: (i, 0)),
            pl.BlockSpec((D, H), lambda i: (0, 0)),
        ],
        out_specs=pl.BlockSpec((_BLK, H), lambda i: (i, 0)),
        out_shape=jax.ShapeDtypeStruct((N, H), jnp.float32),
    )(x, W_neigh1)

    deg1 = _make_sc_deg(n_acc, nch)(dst_r)
    acc1 = _make_sc_agg(n_acc, H, nch_a, nch_b)(m1, src_a, dst_a, src_b, dst_b)

    h1, m2, rdeg = pl.pallas_call(
        _mid_body,
        grid=(grid,),
        in_specs=[
            pl.BlockSpec((_BLK, D), lambda i: (i, 0)),
            pl.BlockSpec((2, _BLK, H), lambda i: (0, i, 0)),
            pl.BlockSpec((2, _BLK, 128), lambda i: (0, i, 0)),
            pl.BlockSpec((D, H), lambda i: (0, 0)),
            pl.BlockSpec((1, H), lambda i: (0, 0)),
            pl.BlockSpec((H, H), lambda i: (0, 0)),
        ],
        out_specs=[
            pl.BlockSpec((_BLK, H), lambda i: (i, 0)),
            pl.BlockSpec((_BLK, H), lambda i: (i, 0)),
            pl.BlockSpec((_BLK, H), lambda i: (i, 0)),
        ],
        out_shape=[
            jax.ShapeDtypeStruct((N, H), jnp.float32),
            jax.ShapeDtypeStruct((N, H), jnp.float32),
            jax.ShapeDtypeStruct((N, H), jnp.float32),
        ],
    )(x, acc1, deg1, W_self1, b1.reshape(1, H), W_neigh2)

    acc2 = _make_sc_agg(n_acc, H, nch_a, nch_b)(m2, src_a, dst_a, src_b, dst_b)

    out = pl.pallas_call(
        _out_body,
        grid=(grid,),
        in_specs=[
            pl.BlockSpec((_BLK, H), lambda i: (i, 0)),
            pl.BlockSpec((2, _BLK, H), lambda i: (0, i, 0)),
            pl.BlockSpec((_BLK, H), lambda i: (i, 0)),
            pl.BlockSpec((H, H), lambda i: (0, 0)),
            pl.BlockSpec((1, H), lambda i: (0, 0)),
        ],
        out_specs=pl.BlockSpec((_BLK, H), lambda i: (i, 0)),
        out_shape=jax.ShapeDtypeStruct((N, H), jnp.float32),
    )(h1, acc2, rdeg, W_self2, b2.reshape(1, H))
    return out


# speedup vs baseline: 2.1545x; 1.1837x over previous
import functools

import jax
import jax.numpy as jnp
from jax import lax
from jax.experimental import pallas as pl
from jax.experimental.pallas import tpu as pltpu
from jax.experimental.pallas import tpu_sc as plsc

_CHUNK = 128
_FRAC_A = 0.5
_WIN = 16
_NW = 32
_BLK = 1000


def _fill(ref, value, width):
    def _row(i, carry):
        for c in range(width // 16):
            ref[i, pl.ds(c * 16, 16)] = jnp.full((16,), value, jnp.float32)
        return carry

    lax.fori_loop(0, _CHUNK, _row, 0)


def _row_blocks(rpt):
    off = 0
    while off < rpt:
        sz = min(_CHUNK, rpt - off)
        yield off, sz
        off += sz


_MESH = plsc.VectorSubcoreMesh(
    core_axis_name="c", subcore_axis_name="s", num_cores=2, num_subcores=16)


def _make_sc_agg(n_acc, width, nch_a, nch_b):
    rpt = n_acc // 16
    nch = max(nch_a, nch_b)

    @functools.partial(
        pl.kernel,
        out_type=jax.ShapeDtypeStruct((2, n_acc, width), jnp.float32),
        mesh=_MESH,
        scratch_types=[
            pltpu.VMEM((nch, _CHUNK), jnp.int32),
            pltpu.VMEM((nch, _CHUNK), jnp.int32),
            pltpu.VMEM((_CHUNK, width), jnp.float32),
            pltpu.VMEM_SHARED((n_acc, width), jnp.float32),
            pltpu.SemaphoreType.DMA,
        ],
    )
    def sc_agg(table, src_a, dst_a, src_b, dst_b, out,
               src_v, dst_v, rows_a, acc, sem_a):
        cid = lax.axis_index("c")
        sid = lax.axis_index("s")
        base = sid * rpt

        _fill(rows_a, 0.0, width)
        for off, sz in _row_blocks(rpt):
            pltpu.sync_copy(rows_a.at[pl.ds(0, sz)],
                            acc.at[pl.ds(base + off, sz)])
        plsc.subcore_barrier()

        @pl.when(cid == 0)
        def _():
            pltpu.sync_copy(src_a.at[sid], src_v.at[pl.ds(0, nch_a)])
            pltpu.sync_copy(dst_a.at[sid], dst_v.at[pl.ds(0, nch_a)])

        @pl.when(cid == 1)
        def _():
            pltpu.sync_copy(src_b.at[sid], src_v.at[pl.ds(0, nch_b)])
            pltpu.sync_copy(dst_b.at[sid], dst_v.at[pl.ds(0, nch_b)])

        def _chunk(j, c_):
            pltpu.async_copy(table.at[src_v.at[j]], rows_a, sem_a).wait()
            pltpu.sync_copy(rows_a, acc.at[dst_v.at[j]], add=True)
            return c_

        lax.fori_loop(0, jnp.where(cid == 0, nch_a, nch_b), _chunk, 0)
        plsc.subcore_barrier()

        for off, sz in _row_blocks(rpt):
            pltpu.sync_copy(acc.at[pl.ds(base + off, sz)],
                            rows_a.at[pl.ds(0, sz)])
            pltpu.sync_copy(rows_a.at[pl.ds(0, sz)],
                            out.at[cid, pl.ds(base + off, sz)])

    return sc_agg


def _make_sc_deg(n_acc, nch, dw=128):
    rpt = n_acc // 16

    @functools.partial(
        pl.kernel,
        out_type=jax.ShapeDtypeStruct((2, n_acc, dw), jnp.float32),
        mesh=_MESH,
        scratch_types=[
            pltpu.VMEM((nch, _CHUNK), jnp.int32),
            pltpu.VMEM((_CHUNK, dw), jnp.float32),
            pltpu.VMEM_SHARED((n_acc, dw), jnp.float32),
        ],
    )
    def sc_deg(dst_idx, out, dst_v, ones_v, dacc):
        cid = lax.axis_index("c")
        sid = lax.axis_index("s")
        wid = sid * 2 + cid
        base = sid * rpt

        _fill(ones_v, 0.0, dw)
        for off, sz in _row_blocks(rpt):
            pltpu.sync_copy(ones_v.at[pl.ds(0, sz)],
                            dacc.at[pl.ds(base + off, sz)])
        _fill(ones_v, 1.0, dw)
        plsc.subcore_barrier()

        pltpu.sync_copy(dst_idx.at[wid], dst_v)

        def _edge_chunk(j, carry):
            pltpu.sync_copy(ones_v, dacc.at[dst_v.at[j]], add=True)
            return carry

        lax.fori_loop(0, nch, _edge_chunk, 0)
        plsc.subcore_barrier()

        for off, sz in _row_blocks(rpt):
            pltpu.sync_copy(dacc.at[pl.ds(base + off, sz)],
                            ones_v.at[pl.ds(0, sz)])
            pltpu.sync_copy(ones_v.at[pl.ds(0, sz)],
                            out.at[cid, pl.ds(base + off, sz)])

    return sc_deg


def _table1_body(x_ref, w_ref, o_ref):
    o_ref[...] = jnp.dot(
        x_ref[...], w_ref[...], preferred_element_type=jnp.float32)


def _mid_body(x_ref, acc_ref, deg_ref, ws_ref, b_ref, wn2_ref,
              h1_ref, m2_ref, rd_ref):
    agg = acc_ref[0] + acc_ref[1]
    deg = (deg_ref[0] + deg_ref[1])[:, 0:1]
    rdeg = 1.0 / jnp.maximum(deg, 1.0)
    h = jnp.dot(x_ref[...], ws_ref[...], preferred_element_type=jnp.float32)
    h = jnp.maximum(h + agg * rdeg + b_ref[...], 0.0)
    h1_ref[...] = h
    m2_ref[...] = jnp.dot(h, wn2_ref[...], preferred_element_type=jnp.float32)
    rd_ref[...] = jnp.broadcast_to(rdeg, rd_ref.shape)


def _out_body(h1_ref, acc_ref, rd_ref, ws_ref, b_ref, o_ref):
    s = acc_ref[0] + acc_ref[1]
    o = jnp.dot(h1_ref[...], ws_ref[...], preferred_element_type=jnp.float32)
    o_ref[...] = o + s * rd_ref[...] + b_ref[...]


def kernel(x, edge_index, W_self1, W_neigh1, b1, W_self2, W_neigh2, b2):
    N, D = x.shape
    H = W_self1.shape[1]
    E = edge_index.shape[1]
    nch = -(-E // (_NW * _CHUNK))
    e_pad = nch * _NW * _CHUNK
    n_acc = -(-(N + 1) // 2048) * 2048
    grid = N // _BLK

    src = edge_index[0]
    dst = edge_index[1]
    pad = e_pad - E
    pad_src = jnp.arange(pad, dtype=jnp.int32) % N
    src_p = jnp.concatenate([src, pad_src])
    pad_dst = N + (jnp.arange(pad, dtype=jnp.int32) % (n_acc - N))
    dst_p = jnp.concatenate([dst, pad_dst])
    dst_r = dst_p.reshape(_NW, nch, _CHUNK)

    tot_ch = _NW * nch
    nch_a = int(round(tot_ch * _FRAC_A / 16))
    nch_b = tot_ch // 16 - nch_a
    cut = 16 * nch_a * _CHUNK
    src_a = src_p[:cut].reshape(16, nch_a, _CHUNK)
    dst_a = dst_p[:cut].reshape(16, nch_a, _CHUNK)
    src_b = src_p[cut:].reshape(16, nch_b, _CHUNK)
    dst_b = dst_p[cut:].reshape(16, nch_b, _CHUNK)

    m1 = pl.pallas_call(
        _table1_body,
        grid=(grid,),
        in_specs=[
            pl.BlockSpec((_BLK, D), lambda i: (i, 0)),
            pl.BlockSpec((D, H), lambda i: (0, 0)),
        ],
        out_specs=pl.BlockSpec((_BLK, H), lambda i: (i, 0)),
        out_shape=jax.ShapeDtypeStruct((N, H), jnp.float32),
    )(x, W_neigh1)

    deg1 = _make_sc_deg(n_acc, nch)(dst_r)
    acc1 = _make_sc_agg(n_acc, H, nch_a, nch_b)(m1, src_a, dst_a, src_b, dst_b)

    h1, m2, rdeg = pl.pallas_call(
        _mid_body,
        grid=(grid,),
        in_specs=[
            pl.BlockSpec((_BLK, D), lambda i: (i, 0)),
            pl.BlockSpec((2, _BLK, H), lambda i: (0, i, 0)),
            pl.BlockSpec((2, _BLK, 128), lambda i: (0, i, 0)),
            pl.BlockSpec((D, H), lambda i: (0, 0)),
            pl.BlockSpec((1, H), lambda i: (0, 0)),
            pl.BlockSpec((H, H), lambda i: (0, 0)),
        ],
        out_specs=[
            pl.BlockSpec((_BLK, H), lambda i: (i, 0)),
            pl.BlockSpec((_BLK, H), lambda i: (i, 0)),
            pl.BlockSpec((_BLK, H), lambda i: (i, 0)),
        ],
        out_shape=[
            jax.ShapeDtypeStruct((N, H), jnp.float32),
            jax.ShapeDtypeStruct((N, H), jnp.float32),
            jax.ShapeDtypeStruct((N, H), jnp.float32),
        ],
    )(x, acc1, deg1, W_self1, b1.reshape(1, H), W_neigh2)

    acc2 = _make_sc_agg(n_acc, H, nch_a, nch_b)(m2, src_a, dst_a, src_b, dst_b)

    out = pl.pallas_call(
        _out_body,
        grid=(grid,),
        in_specs=[
            pl.BlockSpec((_BLK, H), lambda i: (i, 0)),
            pl.BlockSpec((2, _BLK, H), lambda i: (0, i, 0)),
            pl.BlockSpec((_BLK, H), lambda i: (i, 0)),
            pl.BlockSpec((H, H), lambda i: (0, 0)),
            pl.BlockSpec((1, H), lambda i: (0, 0)),
        ],
        out_specs=pl.BlockSpec((_BLK, H), lambda i: (i, 0)),
        out_shape=jax.ShapeDtypeStruct((N, H), jnp.float32),
    )(h1, acc2, rdeg, W_self2, b2.reshape(1, H))
    return out
